# baseline (speedup 1.0000x reference)
import functools

import jax
import jax.numpy as jnp
from jax import lax
from jax.experimental import pallas as pl
from jax.experimental.pallas import tpu as pltpu
from jax.experimental.pallas import tpu_sc as plsc

N_IN = 16384
N_OUT = 16384
WIDTH = 64
BATCH = 32

NC, NS = 2, 16
NW = NC * NS
JPW = N_OUT // NW
CJ = 16
CHUNKS = JPW // CJ
ROWS = CJ * WIDTH
NSTREAM = 8
SROWS = ROWS // NSTREAM
IDXROWS = N_OUT * WIDTH // SROWS
RPW = IDXROWS // NW
XPW = N_IN // NS
KPT = WIDTH // NS
PIECES = 8
PC = N_IN // PIECES


@functools.partial(
    pl.kernel,
    out_type=[jax.ShapeDtypeStruct((N_OUT, BATCH), jnp.float32),
              jax.ShapeDtypeStruct((NC * WIDTH, N_OUT), jnp.float32)],
    mesh=plsc.VectorSubcoreMesh(core_axis_name="c", subcore_axis_name="s"),
    compiler_params=pltpu.CompilerParams(use_tc_tiling_on_sc=False,
                                         needs_layout_passes=False),
    scratch_types=[
        pltpu.VMEM_SHARED((N_IN, BATCH), jnp.float32),
        pltpu.VMEM((N_IN,), jnp.float32),
        pltpu.VMEM((PC,), jnp.float32),
        pltpu.VMEM((PC,), jnp.float32),
        pltpu.VMEM((PC,), jnp.int32),
        pltpu.VMEM((2, NSTREAM, SROWS), jnp.int32),
        pltpu.VMEM((2, ROWS, BATCH), jnp.float32),
        pltpu.VMEM((2, WIDTH, CJ), jnp.float32),
        pltpu.VMEM((CJ, BATCH), jnp.float32),
        pltpu.SemaphoreType.DMA,
        pltpu.SemaphoreType.DMA,
    ],
)
def _densparse(xt_hbm, wt_hbm, mt_hbm, idx_hbm, out_hbm, ct_hbm,
               xt_sh, w2_v, fm_v, cp_v, mi_v, idx_v, g_v,
               cstr_v, out_v, sem0, sem1):
    tid = lax.axis_index("s")
    scid = lax.axis_index("c")
    wid = tid * NC + scid
    row0 = wid * RPW
    j0 = wid * JPW
    sems = (sem0, sem1)

    pltpu.sync_copy(xt_hbm.at[pl.ds(tid * XPW, XPW)],
                    xt_sh.at[pl.ds(tid * XPW, XPW)])

    for kk in range(KPT):
        k = tid * KPT + kk
        pltpu.sync_copy(wt_hbm.at[k], w2_v)
        for h in range(PIECES):
            pltpu.sync_copy(wt_hbm.at[WIDTH + k, pl.ds(h * PC, PC)], fm_v)

            def mul_body(i, carry):
                a = pl.ds(h * PC + i * 16, 16)
                w2_v[a] = w2_v[a] * fm_v[pl.ds(i * 16, 16)]
                return carry
            lax.fori_loop(0, PC // 16, mul_body, 0)
        for h in range(PIECES):
            pltpu.sync_copy(mt_hbm.at[k, pl.ds(h * PC, PC)], mi_v)
            pltpu.sync_copy(wt_hbm.at[2 * WIDTH + k, pl.ds(h * PC, PC)], fm_v)

            def gat_body(i, carry):
                a = pl.ds(i * 16, 16)
                vals = plsc.load_gather(w2_v, [mi_v[a]])
                cp_v[a] = vals * fm_v[a]
                return carry
            lax.fori_loop(0, PC // 16, gat_body, 0)
            pltpu.sync_copy(cp_v,
                            ct_hbm.at[scid * WIDTH + k, pl.ds(h * PC, PC)])

    plsc.subcore_barrier()

    def fire(c, b):
        pltpu.sync_copy(idx_hbm.at[pl.ds(row0 + c * NSTREAM, NSTREAM)],
                        idx_v.at[b])
        for s in range(NSTREAM):
            pltpu.async_copy(xt_sh.at[idx_v.at[b, s]],
                             g_v.at[b, pl.ds(s * SROWS, SROWS)], sems[b])

    def drain(b):
        for s in range(NSTREAM):
            pltpu.make_async_copy(xt_hbm.at[pl.ds(0, SROWS)],
                                  g_v.at[b, pl.ds(s * SROWS, SROWS)],
                                  sems[b]).wait()

    lane = lax.broadcasted_iota(jnp.int32, (16,), 0)
    gd = lax.GatherDimensionNumbers(offset_dims=(), collapsed_slice_dims=(0,),
                                    start_index_map=(0,))

    def take16(v, idx):
        return lax.gather(v, idx[:, None], gd, (1,),
                          mode=lax.GatherScatterMode.PROMISE_IN_BOUNDS)

    def compute(c, b):
        pltpu.sync_copy(
            ct_hbm.at[pl.ds(scid * WIDTH, WIDTH), pl.ds(j0 + c * CJ, CJ)],
            cstr_v.at[b])

        def jj_body(jj, carry):
            col = jnp.full((16,), 0, jnp.int32) + jj
            cf = [plsc.load_gather(cstr_v.at[b], [lane + 16 * g, col])
                  for g in range(WIDTH // 16)]
            r0 = jj * WIDTH
            acc0 = jnp.zeros((16,), jnp.float32)
            acc1 = jnp.zeros((16,), jnp.float32)
            for k in range(WIDTH):
                cb = take16(cf[k // 16], jnp.full((16,), k % 16, jnp.int32))
                acc0 = acc0 + cb * g_v[b, r0 + k, pl.ds(0, 16)]
                acc1 = acc1 + cb * g_v[b, r0 + k, pl.ds(16, 16)]
            out_v[jj, pl.ds(0, 16)] = acc0
            out_v[jj, pl.ds(16, 16)] = acc1
            return carry
        lax.fori_loop(0, CJ, jj_body, 0)
        pltpu.sync_copy(out_v, out_hbm.at[pl.ds(j0 + c * CJ, CJ)])

    fire(0, 0)

    def outer(g2, carry):
        for b in range(2):
            c = g2 * 2 + b

            @pl.when(c < CHUNKS - 1)
            def _fire_next():
                fire(c + 1, 1 - b)

            drain(b)
            compute(c, b)
        return carry

    lax.fori_loop(0, CHUNKS // 2, outer, 0)


def kernel(x, forward_weights, forward_mask, reverse_mask, output_mapping):
    xt = x.T
    wt = jnp.concatenate(
        [forward_weights, forward_mask, reverse_mask], axis=1).T
    mt = output_mapping.T
    idxf = output_mapping.reshape(IDXROWS, SROWS)
    out, _ = _densparse(xt, wt, mt, idxf)
    return out.T


# speedup vs baseline: 1.1670x; 1.1670x over previous
import functools

import jax
import jax.numpy as jnp
from jax import lax
from jax.experimental import pallas as pl
from jax.experimental.pallas import tpu as pltpu
from jax.experimental.pallas import tpu_sc as plsc

N_IN = 16384
N_OUT = 16384
WIDTH = 64
BATCH = 32

NC, NS = 2, 16
NW = NC * NS
JPW = N_OUT // NW
CJ = 16
CHUNKS = JPW // CJ
ROWS = CJ * WIDTH
NSTREAM = 8
SROWS = ROWS // NSTREAM
IDXROWS = N_OUT * WIDTH // SROWS
RPW = IDXROWS // NW
XPW = N_IN // NS
KPT = WIDTH // NS
PIECES = 16
PC = N_IN // PIECES


@functools.partial(
    pl.kernel,
    out_type=[jax.ShapeDtypeStruct((N_OUT, BATCH), jnp.float32),
              jax.ShapeDtypeStruct((NC * WIDTH, N_OUT), jnp.float32)],
    mesh=plsc.VectorSubcoreMesh(core_axis_name="c", subcore_axis_name="s"),
    compiler_params=pltpu.CompilerParams(use_tc_tiling_on_sc=False,
                                         needs_layout_passes=False),
    scratch_types=[
        pltpu.VMEM_SHARED((N_IN, BATCH), jnp.float32),
        pltpu.VMEM((N_IN,), jnp.float32),
        pltpu.VMEM((2, PC), jnp.float32),
        pltpu.VMEM((2, PC), jnp.float32),
        pltpu.VMEM((2, PC), jnp.int32),
        pltpu.VMEM((2, NSTREAM, SROWS), jnp.int32),
        pltpu.VMEM((2, ROWS, BATCH), jnp.float32),
        pltpu.VMEM((2, WIDTH, CJ), jnp.float32),
        pltpu.VMEM((CJ, BATCH), jnp.float32),
        pltpu.SemaphoreType.DMA,
        pltpu.SemaphoreType.DMA,
    ],
)
def _densparse(xt_hbm, wt_hbm, mt_hbm, idx_hbm, out_hbm, ct_hbm,
               xt_sh, w2_v, fm_v, cp_v, mi_v, idx_v, g_v,
               cstr_v, out_v, sem0, sem1):
    tid = lax.axis_index("s")
    scid = lax.axis_index("c")
    wid = tid * NC + scid
    row0 = wid * RPW
    j0 = wid * JPW
    sems = (sem0, sem1)

    pltpu.sync_copy(xt_hbm.at[pl.ds(tid * XPW, XPW)],
                    xt_sh.at[pl.ds(tid * XPW, XPW)])

    def p1_wait(dst_ref, sem):
        pltpu.make_async_copy(wt_hbm.at[0, pl.ds(0, PC)], dst_ref, sem).wait()

    for kk in range(KPT):
        k = tid * KPT + kk
        pltpu.sync_copy(wt_hbm.at[k], w2_v)
        pltpu.async_copy(wt_hbm.at[WIDTH + k, pl.ds(0, PC)], fm_v.at[0], sem0)

        def mul_piece(h2, carry):
            for b in range(2):
                h = h2 * 2 + b

                @pl.when(h < PIECES - 1)
                def _next():
                    pltpu.async_copy(
                        wt_hbm.at[WIDTH + k, pl.ds((h + 1) * PC, PC)],
                        fm_v.at[1 - b], sem0)

                p1_wait(fm_v.at[b], sem0)

                def mul_body(i, carry2):
                    a = pl.ds(h * PC + i * 16, 16)
                    w2_v[a] = w2_v[a] * fm_v[b, pl.ds(i * 16, 16)]
                    return carry2
                lax.fori_loop(0, PC // 16, mul_body, 0)
            return carry
        lax.fori_loop(0, PIECES // 2, mul_piece, 0)

        pltpu.async_copy(mt_hbm.at[k, pl.ds(0, PC)], mi_v.at[0], sem0)
        pltpu.async_copy(wt_hbm.at[2 * WIDTH + k, pl.ds(0, PC)],
                         fm_v.at[0], sem0)

        def gat_piece(h2, carry):
            for b in range(2):
                h = h2 * 2 + b

                @pl.when(h < PIECES - 1)
                def _next():
                    pltpu.async_copy(mt_hbm.at[k, pl.ds((h + 1) * PC, PC)],
                                     mi_v.at[1 - b], sem0)
                    pltpu.async_copy(
                        wt_hbm.at[2 * WIDTH + k, pl.ds((h + 1) * PC, PC)],
                        fm_v.at[1 - b], sem0)

                p1_wait(mi_v.at[b], sem0)
                p1_wait(fm_v.at[b], sem0)

                @pl.when(h >= 2)
                def _drain_ct():
                    p1_wait(cp_v.at[b], sem1)

                def gat_body(i, carry2):
                    a = pl.ds(i * 16, 16)
                    vals = plsc.load_gather(w2_v, [mi_v[b, a]])
                    cp_v[b, a] = vals * fm_v[b, a]
                    return carry2
                lax.fori_loop(0, PC // 16, gat_body, 0)
                pltpu.async_copy(
                    cp_v.at[b],
                    ct_hbm.at[scid * WIDTH + k, pl.ds(h * PC, PC)], sem1)
            return carry
        lax.fori_loop(0, PIECES // 2, gat_piece, 0)
        p1_wait(cp_v.at[0], sem1)
        p1_wait(cp_v.at[1], sem1)

    plsc.subcore_barrier()

    def fire(c, b):
        pltpu.sync_copy(idx_hbm.at[pl.ds(row0 + c * NSTREAM, NSTREAM)],
                        idx_v.at[b])
        for s in range(NSTREAM):
            pltpu.async_copy(xt_sh.at[idx_v.at[b, s]],
                             g_v.at[b, pl.ds(s * SROWS, SROWS)], sems[b])

    def drain(b):
        for s in range(NSTREAM):
            pltpu.make_async_copy(xt_hbm.at[pl.ds(0, SROWS)],
                                  g_v.at[b, pl.ds(s * SROWS, SROWS)],
                                  sems[b]).wait()

    lane = lax.broadcasted_iota(jnp.int32, (16,), 0)
    gd = lax.GatherDimensionNumbers(offset_dims=(), collapsed_slice_dims=(0,),
                                    start_index_map=(0,))

    def take16(v, idx):
        return lax.gather(v, idx[:, None], gd, (1,),
                          mode=lax.GatherScatterMode.PROMISE_IN_BOUNDS)

    def compute(c, b):
        pltpu.sync_copy(
            ct_hbm.at[pl.ds(scid * WIDTH, WIDTH), pl.ds(j0 + c * CJ, CJ)],
            cstr_v.at[b])

        def jj_body(jj, carry):
            col = jnp.full((16,), 0, jnp.int32) + jj
            cf = [plsc.load_gather(cstr_v.at[b], [lane + 16 * g, col])
                  for g in range(WIDTH // 16)]
            r0 = jj * WIDTH
            acc0 = jnp.zeros((16,), jnp.float32)
            acc1 = jnp.zeros((16,), jnp.float32)
            for k in range(WIDTH):
                cb = take16(cf[k // 16], jnp.full((16,), k % 16, jnp.int32))
                acc0 = acc0 + cb * g_v[b, r0 + k, pl.ds(0, 16)]
                acc1 = acc1 + cb * g_v[b, r0 + k, pl.ds(16, 16)]
            out_v[jj, pl.ds(0, 16)] = acc0
            out_v[jj, pl.ds(16, 16)] = acc1
            return carry
        lax.fori_loop(0, CJ, jj_body, 0)
        pltpu.sync_copy(out_v, out_hbm.at[pl.ds(j0 + c * CJ, CJ)])

    fire(0, 0)

    def outer(g2, carry):
        for b in range(2):
            c = g2 * 2 + b

            @pl.when(c < CHUNKS - 1)
            def _fire_next():
                fire(c + 1, 1 - b)

            drain(b)
            compute(c, b)
        return carry

    lax.fori_loop(0, CHUNKS // 2, outer, 0)


def kernel(x, forward_weights, forward_mask, reverse_mask, output_mapping):
    xt = x.T
    wt = jnp.concatenate(
        [forward_weights, forward_mask, reverse_mask], axis=1).T
    mt = output_mapping.T
    idxf = output_mapping.reshape(IDXROWS, SROWS)
    out, _ = _densparse(xt, wt, mt, idxf)
    return out.T


# speedup vs baseline: 1.2052x; 1.0327x over previous
import functools

import jax
import jax.numpy as jnp
from jax import lax
from jax.experimental import pallas as pl
from jax.experimental.pallas import tpu as pltpu
from jax.experimental.pallas import tpu_sc as plsc

N_IN = 16384
N_OUT = 16384
WIDTH = 64
BATCH = 32

NC, NS = 2, 16
NW = NC * NS
JPW = N_OUT // NW
CJ = 16
CHUNKS = JPW // CJ
ROWS = CJ * WIDTH
NSTREAM = 8
SROWS = ROWS // NSTREAM
IDXROWS = N_OUT * WIDTH // SROWS
RPW = IDXROWS // NW
XPW = N_IN // NS
KPT = WIDTH // NS
PIECES = 16
PC = N_IN // PIECES


@functools.partial(
    pl.kernel,
    out_type=[jax.ShapeDtypeStruct((BATCH, N_OUT), jnp.float32),
              jax.ShapeDtypeStruct((NC * WIDTH, N_OUT), jnp.int32)],
    mesh=plsc.VectorSubcoreMesh(core_axis_name="c", subcore_axis_name="s"),
    compiler_params=pltpu.CompilerParams(use_tc_tiling_on_sc=False,
                                         needs_layout_passes=False),
    scratch_types=[
        pltpu.VMEM_SHARED((N_IN, BATCH), jnp.float32),
        pltpu.VMEM((N_IN,), jnp.int32),
        pltpu.VMEM((2, PC), jnp.int32),
        pltpu.VMEM((2, PC), jnp.int32),
        pltpu.VMEM((2, PC), jnp.int32),
        pltpu.VMEM((2, NSTREAM, SROWS), jnp.int32),
        pltpu.VMEM((2, ROWS, BATCH), jnp.float32),
        pltpu.VMEM((2, WIDTH, CJ), jnp.int32),
        pltpu.VMEM((BATCH, CJ), jnp.float32),
        pltpu.SemaphoreType.DMA,
        pltpu.SemaphoreType.DMA,
    ],
)
def _densparse(xt_hbm, wt_hbm, idx_hbm, out_hbm, ct_hbm,
               xt_sh, w2_v, fm_v, cp_v, mi_v, idx_v, g_v,
               cstr_v, out_v, sem0, sem1):
    def _f32(v):
        return lax.bitcast_convert_type(v, jnp.float32)

    def _i32(v):
        return lax.bitcast_convert_type(v, jnp.int32)

    tid = lax.axis_index("s")
    scid = lax.axis_index("c")
    wid = tid * NC + scid
    row0 = wid * RPW
    j0 = wid * JPW
    sems = (sem0, sem1)

    pltpu.sync_copy(xt_hbm.at[pl.ds(tid * XPW, XPW)],
                    xt_sh.at[pl.ds(tid * XPW, XPW)])

    def p1_wait(dst_ref, sem):
        pltpu.make_async_copy(wt_hbm.at[0, pl.ds(0, PC)], dst_ref, sem).wait()

    for kk in range(KPT):
        k = tid * KPT + kk
        pltpu.sync_copy(wt_hbm.at[k], w2_v)
        pltpu.async_copy(wt_hbm.at[WIDTH + k, pl.ds(0, PC)], fm_v.at[0], sem0)

        def mul_piece(h2, carry):
            for b in range(2):
                h = h2 * 2 + b

                @pl.when(h < PIECES - 1)
                def _next():
                    pltpu.async_copy(
                        wt_hbm.at[WIDTH + k, pl.ds((h + 1) * PC, PC)],
                        fm_v.at[1 - b], sem0)

                p1_wait(fm_v.at[b], sem0)

                def mul_body(i, carry2):
                    a = pl.ds(h * PC + i * 16, 16)
                    prod = (_f32(w2_v[a]) * _f32(fm_v[b, pl.ds(i * 16, 16)]))
                    w2_v[a] = _i32(prod)
                    return carry2
                lax.fori_loop(0, PC // 16, mul_body, 0)
            return carry
        lax.fori_loop(0, PIECES // 2, mul_piece, 0)

        pltpu.async_copy(wt_hbm.at[3 * WIDTH + k, pl.ds(0, PC)],
                         mi_v.at[0], sem0)
        pltpu.async_copy(wt_hbm.at[2 * WIDTH + k, pl.ds(0, PC)],
                         fm_v.at[0], sem0)

        def gat_piece(h2, carry):
            for b in range(2):
                h = h2 * 2 + b

                @pl.when(h < PIECES - 1)
                def _next():
                    pltpu.async_copy(
                        wt_hbm.at[3 * WIDTH + k, pl.ds((h + 1) * PC, PC)],
                        mi_v.at[1 - b], sem0)
                    pltpu.async_copy(
                        wt_hbm.at[2 * WIDTH + k, pl.ds((h + 1) * PC, PC)],
                        fm_v.at[1 - b], sem0)

                p1_wait(mi_v.at[b], sem0)
                p1_wait(fm_v.at[b], sem0)

                @pl.when(h >= 2)
                def _drain_ct():
                    p1_wait(cp_v.at[b], sem1)

                def gat_body(i, carry2):
                    a = pl.ds(i * 16, 16)
                    vals = plsc.load_gather(w2_v, [mi_v[b, a]])
                    cp_v[b, a] = _i32(_f32(vals) * _f32(fm_v[b, a]))
                    return carry2
                lax.fori_loop(0, PC // 16, gat_body, 0)
                pltpu.async_copy(
                    cp_v.at[b],
                    ct_hbm.at[scid * WIDTH + k, pl.ds(h * PC, PC)], sem1)
            return carry
        lax.fori_loop(0, PIECES // 2, gat_piece, 0)
        p1_wait(cp_v.at[0], sem1)
        p1_wait(cp_v.at[1], sem1)

    plsc.subcore_barrier()

    def fire(c, b):
        pltpu.sync_copy(idx_hbm.at[pl.ds(row0 + c * NSTREAM, NSTREAM)],
                        idx_v.at[b])
        for s in range(NSTREAM):
            pltpu.async_copy(xt_sh.at[idx_v.at[b, s]],
                             g_v.at[b, pl.ds(s * SROWS, SROWS)], sems[b])

    def drain(b):
        for s in range(NSTREAM):
            pltpu.make_async_copy(xt_hbm.at[pl.ds(0, SROWS)],
                                  g_v.at[b, pl.ds(s * SROWS, SROWS)],
                                  sems[b]).wait()

    lane = lax.broadcasted_iota(jnp.int32, (16,), 0)
    gd = lax.GatherDimensionNumbers(offset_dims=(), collapsed_slice_dims=(0,),
                                    start_index_map=(0,))

    def take16(v, idx):
        return lax.gather(v, idx[:, None], gd, (1,),
                          mode=lax.GatherScatterMode.PROMISE_IN_BOUNDS)

    def compute(c, b):
        pltpu.sync_copy(
            ct_hbm.at[pl.ds(scid * WIDTH, WIDTH), pl.ds(j0 + c * CJ, CJ)],
            cstr_v.at[b])

        def jj_body(jj, carry):
            col = jnp.full((16,), 0, jnp.int32) + jj
            cf = [_f32(plsc.load_gather(cstr_v.at[b], [lane + 16 * g, col]))
                  for g in range(WIDTH // 16)]
            r0 = jj * WIDTH
            acc0 = jnp.zeros((16,), jnp.float32)
            acc1 = jnp.zeros((16,), jnp.float32)
            for k in range(WIDTH):
                cb = take16(cf[k // 16], jnp.full((16,), k % 16, jnp.int32))
                acc0 = acc0 + cb * g_v[b, r0 + k, pl.ds(0, 16)]
                acc1 = acc1 + cb * g_v[b, r0 + k, pl.ds(16, 16)]
            plsc.store_scatter(out_v, [lane, col], acc0)
            plsc.store_scatter(out_v, [lane + 16, col], acc1)
            return carry
        lax.fori_loop(0, CJ, jj_body, 0)
        pltpu.sync_copy(out_v, out_hbm.at[:, pl.ds(j0 + c * CJ, CJ)])

    fire(0, 0)

    def outer(g2, carry):
        for b in range(2):
            c = g2 * 2 + b

            @pl.when(c < CHUNKS - 1)
            def _fire_next():
                fire(c + 1, 1 - b)

            drain(b)
            compute(c, b)
        return carry

    lax.fori_loop(0, CHUNKS // 2, outer, 0)


def kernel(x, forward_weights, forward_mask, reverse_mask, output_mapping):
    xt = x.T
    bc = lambda a: jax.lax.bitcast_convert_type(a, jnp.int32)
    wt = jnp.concatenate(
        [bc(forward_weights), bc(forward_mask), bc(reverse_mask),
         output_mapping], axis=1).T
    idxf = output_mapping.reshape(IDXROWS, SROWS)
    out, _ = _densparse(xt, wt, idxf)
    return out
